# Initial kernel scaffold; baseline (speedup 1.0000x reference)
import jax
import jax.numpy as jnp
from jax.experimental import pallas as pl


def kernel(x, table):
    raise NotImplementedError("write your pallas kernel here")


# speedup vs baseline: 1.0162x; 1.0162x over previous
import functools
import math

import jax
import jax.numpy as jnp
from jax import lax
from jax.experimental import pallas as pl
from jax.experimental.pallas import tpu as pltpu
from jax.experimental.pallas import tpu_sc as plsc

D_MODEL = 32
SCALE = math.sqrt(float(D_MODEL))

NUM_CORES = 2
NUM_SUBCORES = 16
NUM_WORKERS = NUM_CORES * NUM_SUBCORES

GRANULE = 128
GRANULES_PER_CHUNK = 8
CHUNK = GRANULE * GRANULES_PER_CHUNK


def _make_kernel(batch: int):
    assert batch % (NUM_WORKERS * CHUNK) == 0
    rows_per_worker = batch // NUM_WORKERS
    chunks_per_worker = rows_per_worker // CHUNK
    granules_total = batch // GRANULE

    mesh = plsc.VectorSubcoreMesh(
        core_axis_name="c", subcore_axis_name="s",
        num_cores=NUM_CORES, num_subcores=NUM_SUBCORES)

    @functools.partial(
        pl.kernel,
        out_type=jax.ShapeDtypeStruct((batch, D_MODEL), jnp.float32),
        mesh=mesh,
        scratch_types=[
            pltpu.VMEM((GRANULES_PER_CHUNK, GRANULE), jnp.int32),
            pltpu.VMEM((CHUNK, D_MODEL), jnp.float32),
            pltpu.SemaphoreType.DMA,
        ],
        compiler_params=pltpu.CompilerParams(use_tc_tiling_on_sc=False),
    )
    def emb_kernel(idx_hbm, table_hbm, out_hbm, idx_v, rows_v, sem):
        wid = lax.axis_index("s") * NUM_CORES + lax.axis_index("c")
        row_base = wid * rows_per_worker
        gran_base = row_base // GRANULE

        def chunk_body(ci):
            gstart = pl.multiple_of(
                gran_base + ci * GRANULES_PER_CHUNK, GRANULES_PER_CHUNK)
            pltpu.sync_copy(
                idx_hbm.at[pl.ds(gstart, GRANULES_PER_CHUNK)], idx_v)
            descs = []
            for g in range(GRANULES_PER_CHUNK):
                descs.append(pltpu.async_copy(
                    table_hbm.at[idx_v.at[g]],
                    rows_v.at[pl.ds(g * GRANULE, GRANULE)],
                    sem))
            for d in descs:
                d.wait()

            def scale_body(i):
                rows_v[i, pl.ds(0, 16)] = rows_v[i, pl.ds(0, 16)] * SCALE
                rows_v[i, pl.ds(16, 16)] = rows_v[i, pl.ds(16, 16)] * SCALE
            plsc.parallel_loop(0, CHUNK, 1, unroll=8)(scale_body)

            pltpu.sync_copy(
                rows_v, out_hbm.at[pl.ds(row_base + ci * CHUNK, CHUNK)])

        pl.loop(0, chunks_per_worker)(chunk_body)

    return emb_kernel


@jax.jit
def kernel(x, table):
    batch = x.shape[0] * x.shape[1]
    idx = x.reshape(batch // GRANULE, GRANULE).astype(jnp.int32)
    out = _make_kernel(batch)(idx, table)
    return out.reshape(x.shape[0], x.shape[1], D_MODEL)


# speedup vs baseline: 1.6207x; 1.5949x over previous
import functools
import math

import jax
import jax.numpy as jnp
from jax import lax
from jax.experimental import pallas as pl
from jax.experimental.pallas import tpu as pltpu
from jax.experimental.pallas import tpu_sc as plsc

D_MODEL = 32
SCALE = math.sqrt(float(D_MODEL))

NUM_CORES = 2
NUM_SUBCORES = 16
NUM_WORKERS = NUM_CORES * NUM_SUBCORES

ROWS_PER_CHUNK = 16


def _make_kernel(nrows: int, ncols: int):
    assert nrows % (NUM_WORKERS * ROWS_PER_CHUNK) == 0
    rows_per_worker = nrows // NUM_WORKERS
    chunks_per_worker = rows_per_worker // ROWS_PER_CHUNK

    mesh = plsc.VectorSubcoreMesh(
        core_axis_name="c", subcore_axis_name="s",
        num_cores=NUM_CORES, num_subcores=NUM_SUBCORES)

    @functools.partial(
        pl.kernel,
        out_type=jax.ShapeDtypeStruct((nrows, ncols, D_MODEL), jnp.float32),
        mesh=mesh,
        scratch_types=[
            pltpu.VMEM((rows_per_worker, ncols), jnp.int32),
            pltpu.VMEM((ROWS_PER_CHUNK, ncols, D_MODEL), jnp.float32),
            pltpu.SemaphoreType.DMA,
        ],
        compiler_params=pltpu.CompilerParams(use_tc_tiling_on_sc=False),
    )
    def emb_kernel(x_hbm, table_hbm, out_hbm, idx_v, rows_v, sem):
        wid = lax.axis_index("s") * NUM_CORES + lax.axis_index("c")
        row_base = wid * rows_per_worker

        pltpu.sync_copy(
            x_hbm.at[pl.ds(pl.multiple_of(row_base, rows_per_worker),
                           rows_per_worker)],
            idx_v)

        def chunk_body(ci):
            descs = []
            for r in range(ROWS_PER_CHUNK):
                descs.append(pltpu.async_copy(
                    table_hbm.at[idx_v.at[ci * ROWS_PER_CHUNK + r]],
                    rows_v.at[r],
                    sem))
            for d in descs:
                d.wait()

            for r in range(ROWS_PER_CHUNK):
                def scale_body(c, _r=r):
                    rows_v[_r, c, pl.ds(0, 16)] = (
                        rows_v[_r, c, pl.ds(0, 16)] * SCALE)
                    rows_v[_r, c, pl.ds(16, 16)] = (
                        rows_v[_r, c, pl.ds(16, 16)] * SCALE)
                plsc.parallel_loop(0, ncols, 1, unroll=5)(scale_body)

            out_start = pl.multiple_of(
                row_base + ci * ROWS_PER_CHUNK, ROWS_PER_CHUNK)
            pltpu.sync_copy(rows_v, out_hbm.at[pl.ds(out_start, ROWS_PER_CHUNK)])

        pl.loop(0, chunks_per_worker)(chunk_body)

    return emb_kernel


@jax.jit
def kernel(x, table):
    return _make_kernel(x.shape[0], x.shape[1])(x.astype(jnp.int32), table)


# speedup vs baseline: 1.7226x; 1.0628x over previous
import functools
import math

import jax
import jax.numpy as jnp
from jax import lax
from jax.experimental import pallas as pl
from jax.experimental.pallas import tpu as pltpu
from jax.experimental.pallas import tpu_sc as plsc

D_MODEL = 32
SCALE = math.sqrt(float(D_MODEL))

NUM_CORES = 2
NUM_SUBCORES = 16
NUM_WORKERS = NUM_CORES * NUM_SUBCORES

LANES = 16
TILE_B = 128
SUBL = 8
TR = D_MODEL // SUBL


def _make_kernel(nrows: int, ncols: int):
    rows_per_worker = nrows // NUM_WORKERS
    btiles_per_worker = rows_per_worker // TILE_B
    assert nrows == NUM_WORKERS * rows_per_worker
    assert rows_per_worker % TILE_B == 0

    mesh = plsc.VectorSubcoreMesh(
        core_axis_name="c", subcore_axis_name="s",
        num_cores=NUM_CORES, num_subcores=NUM_SUBCORES)

    @functools.partial(
        pl.kernel,
        out_type=jax.ShapeDtypeStruct(
            (ncols, TR, nrows // TILE_B, SUBL, TILE_B), jnp.float32),
        mesh=mesh,
        scratch_types=[
            pltpu.VMEM((rows_per_worker, ncols), jnp.int32),
            pltpu.VMEM((ncols, rows_per_worker), jnp.int32),
            pltpu.VMEM((2, rows_per_worker, D_MODEL), jnp.float32),
            pltpu.VMEM((2, TR, btiles_per_worker, SUBL, TILE_B),
                       jnp.float32),
            pltpu.SemaphoreType.DMA,
            pltpu.SemaphoreType.DMA,
        ],
        compiler_params=pltpu.CompilerParams(
            use_tc_tiling_on_sc=False, needs_layout_passes=False),
    )
    def emb_kernel(x_hbm, table_hbm, out_hbm, idx_v, idxT, rows_v, tilebuf,
                   sem_in, sem_out):
        wid = lax.axis_index("s") * NUM_CORES + lax.axis_index("c")
        row_base = wid * rows_per_worker
        iota = lax.iota(jnp.int32, LANES)

        pltpu.sync_copy(
            x_hbm.at[pl.ds(pl.multiple_of(row_base, rows_per_worker),
                           rows_per_worker)],
            idx_v)

        def idx_t_body(t):
            c = t // (rows_per_worker // LANES)
            k = t % (rows_per_worker // LANES)
            b_idx = k * LANES + iota
            c_idx = jnp.full((LANES,), c, jnp.int32)
            idxT[c, pl.ds(k * LANES, LANES)] = plsc.load_gather(
                idx_v, [b_idx, c_idx])
        plsc.parallel_loop(0, ncols * (rows_per_worker // LANES), 1,
                           unroll=4)(idx_t_body)

        def fire(c, buf):
            for k in range(btiles_per_worker):
                pltpu.async_copy(
                    table_hbm.at[idxT.at[c, pl.ds(k * TILE_B, TILE_B)]],
                    rows_v.at[buf, pl.ds(k * TILE_B, TILE_B)],
                    sem_in)

        def drain_gathers(buf):
            for k in range(btiles_per_worker):
                pltpu.make_async_copy(
                    table_hbm.at[idxT.at[0, pl.ds(0, TILE_B)]],
                    rows_v.at[buf, pl.ds(k * TILE_B, TILE_B)],
                    sem_in).wait()

        def wait_out(buf):
            pltpu.make_async_copy(
                tilebuf.at[buf],
                out_hbm.at[0, :, pl.ds(0, btiles_per_worker)],
                sem_out).wait()

        lines_per_tr = btiles_per_worker * SUBL * (TILE_B // LANES)

        def process(c, buf):
            for tr in range(TR):
                def tp_body(t, _tr=tr):
                    tc = t // (SUBL * (TILE_B // LANES))
                    r1 = t % (SUBL * (TILE_B // LANES))
                    s = r1 // (TILE_B // LANES)
                    lc = r1 % (TILE_B // LANES)
                    b_idx = tc * TILE_B + lc * LANES + iota
                    d_idx = jnp.full((LANES,), _tr * SUBL + s, jnp.int32)
                    v = plsc.load_gather(rows_v.at[buf], [b_idx, d_idx])
                    tilebuf[buf, _tr, tc, s, pl.ds(lc * LANES, LANES)] = (
                        v * SCALE)
                plsc.parallel_loop(0, lines_per_tr, 1, unroll=4)(tp_body)
            pltpu.async_copy(
                tilebuf.at[buf],
                out_hbm.at[c, :, pl.ds(pl.multiple_of(
                    wid * btiles_per_worker, btiles_per_worker),
                    btiles_per_worker)],
                sem_out)

        fire(0, 0)

        def pair_body(kk):
            for bb in (0, 1):
                c = 2 * kk + bb

                @pl.when(c < ncols - 1)
                def _():
                    fire(c + 1, 1 - bb)

                drain_gathers(bb)

                @pl.when(c >= 2)
                def _():
                    wait_out(bb)

                process(c, bb)

        pl.loop(0, ncols // 2)(pair_body)

        wait_out(0)
        wait_out(1)

    return emb_kernel


@jax.jit
def kernel(x, table):
    nrows, ncols = x.shape
    out5 = _make_kernel(nrows, ncols)(x.astype(jnp.int32), table)
    return out5.transpose(2, 4, 0, 1, 3).reshape(nrows, ncols, D_MODEL)


# speedup vs baseline: 1.7800x; 1.0333x over previous
import functools
import math

import jax
import jax.numpy as jnp
from jax import lax
from jax.experimental import pallas as pl
from jax.experimental.pallas import tpu as pltpu
from jax.experimental.pallas import tpu_sc as plsc

D_MODEL = 32
SCALE = math.sqrt(float(D_MODEL))

NUM_CORES = 2
NUM_SUBCORES = 16
NUM_WORKERS = NUM_CORES * NUM_SUBCORES

LANES = 16
TILE_B = 128
SUBL = 8
TR = D_MODEL // SUBL


def _make_kernel(nrows: int, ncols: int):
    rows_per_worker = nrows // NUM_WORKERS
    btiles_per_worker = rows_per_worker // TILE_B
    assert nrows == NUM_WORKERS * rows_per_worker
    assert rows_per_worker % TILE_B == 0

    mesh = plsc.VectorSubcoreMesh(
        core_axis_name="c", subcore_axis_name="s",
        num_cores=NUM_CORES, num_subcores=NUM_SUBCORES)

    @functools.partial(
        pl.kernel,
        out_type=jax.ShapeDtypeStruct(
            (ncols, TR, nrows // TILE_B, SUBL, TILE_B), jnp.float32),
        mesh=mesh,
        scratch_types=[
            pltpu.VMEM((rows_per_worker, ncols), jnp.int32),
            pltpu.VMEM((ncols, rows_per_worker), jnp.int32),
            pltpu.VMEM((2, rows_per_worker, D_MODEL), jnp.float32),
            pltpu.VMEM((2, TR, btiles_per_worker, SUBL, TILE_B),
                       jnp.float32),
            pltpu.SemaphoreType.DMA,
            pltpu.SemaphoreType.DMA,
        ],
        compiler_params=pltpu.CompilerParams(
            use_tc_tiling_on_sc=False, needs_layout_passes=False),
    )
    def emb_kernel(x_hbm, table_hbm, out_hbm, idx_v, idxT, rows_v, tilebuf,
                   sem_in, sem_out):
        wid = lax.axis_index("s") * NUM_CORES + lax.axis_index("c")
        row_base = wid * rows_per_worker
        iota = lax.iota(jnp.int32, LANES)

        pltpu.sync_copy(
            x_hbm.at[pl.ds(pl.multiple_of(row_base, rows_per_worker),
                           rows_per_worker)],
            idx_v)

        def idx_t_body(t):
            c = t // (rows_per_worker // LANES)
            k = t % (rows_per_worker // LANES)
            b_idx = k * LANES + iota
            c_idx = jnp.full((LANES,), c, jnp.int32)
            idxT[c, pl.ds(k * LANES, LANES)] = plsc.load_gather(
                idx_v, [b_idx, c_idx])
        plsc.parallel_loop(0, ncols * (rows_per_worker // LANES), 1,
                           unroll=4)(idx_t_body)

        def fire(c, buf):
            for k in range(btiles_per_worker):
                pltpu.async_copy(
                    table_hbm.at[idxT.at[c, pl.ds(k * TILE_B, TILE_B)]],
                    rows_v.at[buf, pl.ds(k * TILE_B, TILE_B)],
                    sem_in)

        def drain_gathers(buf):
            for k in range(btiles_per_worker):
                pltpu.make_async_copy(
                    table_hbm.at[idxT.at[0, pl.ds(0, TILE_B)]],
                    rows_v.at[buf, pl.ds(k * TILE_B, TILE_B)],
                    sem_in).wait()

        def wait_out(buf):
            pltpu.make_async_copy(
                tilebuf.at[buf],
                out_hbm.at[0, :, pl.ds(0, btiles_per_worker)],
                sem_out).wait()

        lchunks = TILE_B // LANES

        def process(c, buf):
            for tr in range(TR):
                for tc in range(btiles_per_worker):
                    def tp_body(t, _tr=tr, _tc=tc):
                        s = lax.shift_right_logical(t, 3)
                        lc = lax.bitwise_and(t, 7)
                        b_idx = (_tc * TILE_B + lax.shift_left(lc, 4)) + iota
                        d_idx = jnp.full((LANES,), _tr * SUBL, jnp.int32) + s
                        v = plsc.load_gather(rows_v.at[buf], [b_idx, d_idx])
                        tilebuf[buf, _tr, _tc, s,
                                pl.ds(lax.shift_left(lc, 4), LANES)] = (
                            v * SCALE)
                    plsc.parallel_loop(0, SUBL * lchunks, 1,
                                       unroll=8)(tp_body)
            pltpu.async_copy(
                tilebuf.at[buf],
                out_hbm.at[c, :, pl.ds(pl.multiple_of(
                    wid * btiles_per_worker, btiles_per_worker),
                    btiles_per_worker)],
                sem_out)

        fire(0, 0)

        def pair_body(kk):
            for bb in (0, 1):
                c = 2 * kk + bb

                @pl.when(c < ncols - 1)
                def _():
                    fire(c + 1, 1 - bb)

                drain_gathers(bb)

                @pl.when(c >= 2)
                def _():
                    wait_out(bb)

                process(c, bb)

        pl.loop(0, ncols // 2)(pair_body)

        wait_out(0)
        wait_out(1)

    return emb_kernel


@jax.jit
def kernel(x, table):
    nrows, ncols = x.shape
    out5 = _make_kernel(nrows, ncols)(x.astype(jnp.int32), table)
    return out5.transpose(2, 4, 0, 1, 3).reshape(nrows, ncols, D_MODEL)


# speedup vs baseline: 2.7979x; 1.5718x over previous
import functools
import math

import jax
import jax.numpy as jnp
from jax import lax
from jax.experimental import pallas as pl
from jax.experimental.pallas import tpu as pltpu
from jax.experimental.pallas import tpu_sc as plsc

D_MODEL = 32
SCALE = math.sqrt(float(D_MODEL))

NUM_CORES = 2
NUM_SUBCORES = 16
NUM_WORKERS = NUM_CORES * NUM_SUBCORES

LANES = 16
TILE_B = 128
SUBL = 8
TR = D_MODEL // SUBL


def _make_kernel(nrows: int, ncols: int):
    rows_per_worker = nrows // NUM_WORKERS
    btiles_per_worker = rows_per_worker // TILE_B
    assert nrows == NUM_WORKERS * rows_per_worker
    assert rows_per_worker % TILE_B == 0

    mesh = plsc.VectorSubcoreMesh(
        core_axis_name="c", subcore_axis_name="s",
        num_cores=NUM_CORES, num_subcores=NUM_SUBCORES)

    @functools.partial(
        pl.kernel,
        out_type=jax.ShapeDtypeStruct(
            (ncols, TR, nrows // TILE_B, SUBL, TILE_B), jnp.float32),
        mesh=mesh,
        scratch_types=[
            pltpu.VMEM((rows_per_worker, ncols), jnp.int32),
            pltpu.VMEM((ncols, rows_per_worker), jnp.int32),
            pltpu.VMEM((2, rows_per_worker, D_MODEL), jnp.float32),
            pltpu.VMEM((2, TR, btiles_per_worker, SUBL, TILE_B),
                       jnp.float32),
            pltpu.SemaphoreType.DMA,
            pltpu.SemaphoreType.DMA,
        ],
        compiler_params=pltpu.CompilerParams(
            use_tc_tiling_on_sc=False, needs_layout_passes=False),
    )
    def emb_kernel(x_hbm, table_hbm, out_hbm, idx_v, idxT, rows_v, tilebuf,
                   sem_in, sem_out):
        wid = lax.axis_index("s") * NUM_CORES + lax.axis_index("c")
        row_base = wid * rows_per_worker
        iota = lax.iota(jnp.int32, LANES)

        pltpu.sync_copy(
            x_hbm.at[pl.ds(pl.multiple_of(row_base, rows_per_worker),
                           rows_per_worker)],
            idx_v)

        def idx_t_body(t):
            c = t // (rows_per_worker // LANES)
            k = t % (rows_per_worker // LANES)
            b_idx = k * LANES + iota
            c_idx = jnp.full((LANES,), c, jnp.int32)
            idxT[c, pl.ds(k * LANES, LANES)] = plsc.load_gather(
                idx_v, [b_idx, c_idx])
        plsc.parallel_loop(0, ncols * (rows_per_worker // LANES), 1,
                           unroll=4)(idx_t_body)

        def fire(c, buf):
            for k in range(btiles_per_worker):
                pltpu.async_copy(
                    table_hbm.at[idxT.at[c, pl.ds(k * TILE_B, TILE_B)]],
                    rows_v.at[buf, pl.ds(k * TILE_B, TILE_B)],
                    sem_in)

        def drain_gathers(buf):
            for k in range(btiles_per_worker):
                pltpu.make_async_copy(
                    table_hbm.at[idxT.at[0, pl.ds(0, TILE_B)]],
                    rows_v.at[buf, pl.ds(k * TILE_B, TILE_B)],
                    sem_in).wait()

        def wait_out(buf):
            pltpu.make_async_copy(
                tilebuf.at[buf],
                out_hbm.at[0, :, pl.ds(0, btiles_per_worker)],
                sem_out).wait()

        lchunks = TILE_B // LANES

        def process(c, buf):
            for tr in range(TR):
                for tc in range(btiles_per_worker):
                    def tp_body(t, _tr=tr, _tc=tc):
                        sh = lax.shift_right_logical(t, 3)
                        lc = lax.bitwise_and(t, 7)
                        s_idx = lax.bitwise_and(iota + sh, 7)
                        d_idx = s_idx + (_tr * SUBL)
                        l_idx = lax.shift_left(lc, 4) + iota
                        b_idx = l_idx + (_tc * TILE_B)
                        v = plsc.load_gather(rows_v.at[buf], [b_idx, d_idx])
                        plsc.store_scatter(
                            tilebuf.at[buf, _tr, _tc], [s_idx, l_idx],
                            v * SCALE)
                    plsc.parallel_loop(0, SUBL * lchunks, 1,
                                       unroll=8)(tp_body)
            pltpu.async_copy(
                tilebuf.at[buf],
                out_hbm.at[c, :, pl.ds(pl.multiple_of(
                    wid * btiles_per_worker, btiles_per_worker),
                    btiles_per_worker)],
                sem_out)

        fire(0, 0)

        def pair_body(kk):
            for bb in (0, 1):
                c = 2 * kk + bb

                @pl.when(c < ncols - 1)
                def _():
                    fire(c + 1, 1 - bb)

                drain_gathers(bb)

                @pl.when(c >= 2)
                def _():
                    wait_out(bb)

                process(c, bb)

        pl.loop(0, ncols // 2)(pair_body)

        wait_out(0)
        wait_out(1)

    return emb_kernel


@jax.jit
def kernel(x, table):
    nrows, ncols = x.shape
    out5 = _make_kernel(nrows, ncols)(x.astype(jnp.int32), table)
    return out5.transpose(2, 4, 0, 1, 3).reshape(nrows, ncols, D_MODEL)
